# baseline (speedup 1.0000x reference)
import functools

import jax
import jax.numpy as jnp
from jax import lax
from jax.experimental import pallas as pl
from jax.experimental.pallas import tpu as pltpu
from jax.experimental.pallas import tpu_sc as plsc

_VOCAB = 100000
_SEQ = 2048
_BATCH = 4
_D = 128
_ROWS = _BATCH * _SEQ
_NC = 2
_NS = 16
_NW = _NC * _NS
_PPW = _SEQ // _NW
_CH = _PPW
_NCH = _BATCH
_L = 16


def _emb_body(ids_hbm, pos_hbm, tab_hbm, out_hbm, idx_v, rows_v, pos_v,
              psem, fsems, gsems, osems):
    wid = lax.axis_index("s") * _NC + lax.axis_index("c")
    pos_base = wid * _PPW

    pdesc = pltpu.async_copy(pos_hbm.at[pl.ds(pos_base, _PPW)], pos_v, psem)
    fdescs = {
        j: pltpu.async_copy(pos_hbm.at[pl.ds(pos_base, _PPW)], rows_v.at[j],
                            fsems.at[j])
        for j in (2, 3)
    }

    gdescs = []
    for j in range(_NCH):
        pltpu.sync_copy(ids_hbm.at[j, pl.ds(pos_base, _CH)], idx_v.at[j])
        if j >= 2:
            fdescs[j].wait()
        gdescs.append(
            pltpu.async_copy(tab_hbm.at[idx_v.at[j]], rows_v.at[j],
                             gsems.at[j], add=(j >= 2)))
    pdesc.wait()

    gdescs[0].wait()
    gdescs[1].wait()

    def add_row(r, carry):
        for c in range(_D // _L):
            sl = pl.ds(c * _L, _L)
            pv = pos_v[r, sl]
            for j in (0, 1):
                plsc.addupdate(rows_v.at[j, r, sl], pv)
        return carry
    lax.fori_loop(0, _CH, add_row, 0)

    odescs = []
    for j in range(_NCH):
        if j >= 2:
            gdescs[j].wait()
        odescs.append(
            pltpu.async_copy(rows_v.at[j],
                             out_hbm.at[j, pl.ds(pos_base, _CH)],
                             osems.at[j]))
    for d in odescs:
        d.wait()


@jax.jit
def _emb_call(ids, token_embedding, position_embedding):
    mesh = plsc.VectorSubcoreMesh(core_axis_name="c", subcore_axis_name="s")
    run = pl.kernel(
        _emb_body,
        out_type=jax.ShapeDtypeStruct((_BATCH, _SEQ, _D), jnp.float32),
        mesh=mesh,
        scratch_types=[
            pltpu.VMEM((_NCH, _CH), jnp.int32),
            pltpu.VMEM((_NCH, _CH, _D), jnp.float32),
            pltpu.VMEM((_PPW, _D), jnp.float32),
            pltpu.SemaphoreType.DMA,
            pltpu.SemaphoreType.DMA((_NCH,)),
            pltpu.SemaphoreType.DMA((_NCH,)),
            pltpu.SemaphoreType.DMA((_NCH,)),
        ],
    )
    return run(ids, position_embedding, token_embedding)


def kernel(token_ids, token_embedding, position_embedding):
    return _emb_call(token_ids.astype(jnp.int32), token_embedding,
                     position_embedding)


# speedup vs baseline: 1.0151x; 1.0151x over previous
import functools

import jax
import jax.numpy as jnp
from jax import lax
from jax.experimental import pallas as pl
from jax.experimental.pallas import tpu as pltpu
from jax.experimental.pallas import tpu_sc as plsc

_VOCAB = 100000
_SEQ = 2048
_BATCH = 4
_D = 128
_ROWS = _BATCH * _SEQ
_NC = 2
_NS = 16
_NW = _NC * _NS
_PPW = _SEQ // _NW
_CH = _PPW
_NCH = _BATCH
_L = 16


def _emb_body(ids_hbm, pos_hbm, tab_hbm, out_hbm, idx_v, rows_v, pos_v,
              psem, gsems, osems):
    wid = lax.axis_index("s") * _NC + lax.axis_index("c")
    pos_base = wid * _PPW

    pdesc = pltpu.async_copy(pos_hbm.at[pl.ds(pos_base, _PPW)], pos_v, psem)

    gdescs = []
    for j in range(_NCH):
        pltpu.sync_copy(ids_hbm.at[j, pl.ds(pos_base, _CH)], idx_v.at[j])
        gdescs.append(
            pltpu.async_copy(tab_hbm.at[idx_v.at[j]], rows_v.at[j],
                             gsems.at[j]))
    pdesc.wait()

    odescs = []
    for p in range(_NCH // 2):
        js = (2 * p, 2 * p + 1)
        for j in js:
            gdescs[j].wait()

        def _make_add(js):
            def add_row(r):
                for c in range(_D // _L):
                    sl = pl.ds(c * _L, _L)
                    pv = pos_v[r, sl]
                    for j in js:
                        plsc.addupdate(rows_v.at[j, r, sl], pv)
            return add_row
        plsc.parallel_loop(0, _CH)(_make_add(js))
        for j in js:
            odescs.append(
                pltpu.async_copy(rows_v.at[j],
                                 out_hbm.at[j, pl.ds(pos_base, _CH)],
                                 osems.at[j]))
    for d in odescs:
        d.wait()


@jax.jit
def _emb_call(ids, token_embedding, position_embedding):
    mesh = plsc.VectorSubcoreMesh(core_axis_name="c", subcore_axis_name="s")
    run = pl.kernel(
        _emb_body,
        out_type=jax.ShapeDtypeStruct((_BATCH, _SEQ, _D), jnp.float32),
        mesh=mesh,
        scratch_types=[
            pltpu.VMEM((_NCH, _CH), jnp.int32),
            pltpu.VMEM((_NCH, _CH, _D), jnp.float32),
            pltpu.VMEM((_PPW, _D), jnp.float32),
            pltpu.SemaphoreType.DMA,
            pltpu.SemaphoreType.DMA((_NCH,)),
            pltpu.SemaphoreType.DMA((_NCH,)),
        ],
    )
    return run(ids, position_embedding, token_embedding)


def kernel(token_ids, token_embedding, position_embedding):
    return _emb_call(token_ids.astype(jnp.int32), token_embedding,
                     position_embedding)
